# Initial kernel scaffold; baseline (speedup 1.0000x reference)
import jax
import jax.numpy as jnp
from jax.experimental import pallas as pl


def kernel(x, edge_index, W1, b1, W2, b2):
    raise NotImplementedError("write your pallas kernel here")


# speedup vs baseline: 16.7303x; 16.7303x over previous
import functools

import jax
import jax.numpy as jnp
from jax import lax
from jax.experimental import pallas as pl
from jax.experimental.pallas import tpu as pltpu
from jax.experimental.pallas import tpu_sc as plsc

N = 10000
D = 128
E = 320000
CH = E // 128
NC, NS = 2, 16
NW = NC * NS
ROWS_PER_W = CH // NW
EXTRA = CH % NW
SL = 640
SL_LAST = N - SL * (NS - 1)

_mesh = lambda: plsc.VectorSubcoreMesh(core_axis_name="c", subcore_axis_name="s")


def _for_my_slice(s, fn):
    @pl.when(s < NS - 1)
    def _():
        def b(k, carry):
            fn(pl.multiple_of(s * SL + k * 128, 128), 128)
            return carry

        lax.fori_loop(0, SL // 128, b, 0)

    @pl.when(s == NS - 1)
    def _():
        base = (NS - 1) * SL
        for k in range(SL_LAST // 128):
            fn(base + k * 128, 128)
        fn(base + (SL_LAST // 128) * 128, SL_LAST % 128)


def _degree_partials(dst2):
    @functools.partial(
        pl.kernel,
        out_type=jax.ShapeDtypeStruct((NC, 1, N), jnp.float32),
        mesh=_mesh(),
        scratch_types=[
            pltpu.VMEM_SHARED((N,), jnp.float32),
            pltpu.VMEM((128,), jnp.int32),
            pltpu.VMEM((128,), jnp.float32),
            pltpu.VMEM((128,), jnp.float32),
        ],
    )
    def k(dst_hbm, out_hbm, acc, didx, ones_v, stage):
        c = lax.axis_index("c")
        s = lax.axis_index("s")
        w = s * NC + c
        for i in range(8):
            ones_v[pl.ds(i * 16, 16)] = jnp.ones((16,), jnp.float32)
            stage[pl.ds(i * 16, 16)] = jnp.zeros((16,), jnp.float32)

        _for_my_slice(s, lambda off, ln: pltpu.sync_copy(
            stage.at[pl.ds(0, ln)], acc.at[pl.ds(off, ln)]))
        plsc.subcore_barrier()

        base = w * ROWS_PER_W + jnp.minimum(w, EXTRA)
        n = ROWS_PER_W + jnp.where(w < EXTRA, 1, 0)

        def body(j, carry):
            pltpu.sync_copy(dst_hbm.at[base + j], didx)
            pltpu.sync_copy(ones_v, acc.at[didx], add=True)
            return carry

        lax.fori_loop(0, n, body, 0)
        plsc.subcore_barrier()

        def wb(off, ln):
            pltpu.sync_copy(acc.at[pl.ds(off, ln)], stage.at[pl.ds(0, ln)])
            pltpu.sync_copy(stage.at[pl.ds(0, ln)],
                            out_hbm.at[c].at[0].at[pl.ds(off, ln)])

        _for_my_slice(s, wb)

    return k(dst2)


def _propagate(xws, src2, dst2):
    @functools.partial(
        pl.kernel,
        out_type=jax.ShapeDtypeStruct((NC, N, D), jnp.float32),
        mesh=_mesh(),
        scratch_types=[
            pltpu.VMEM_SHARED((N, D), jnp.float32),
            pltpu.VMEM((128,), jnp.int32),
            pltpu.VMEM((128,), jnp.int32),
            pltpu.VMEM((128, D), jnp.float32),
            pltpu.SemaphoreType.DMA,
        ],
    )
    def k(xws_hbm, src_hbm, dst_hbm, out_hbm, acc, sidx, didx, rows, sem):
        c = lax.axis_index("c")
        s = lax.axis_index("s")
        w = s * NC + c

        def zrow(i, carry):
            def zcol(j, c2):
                rows[i, pl.ds(j * 16, 16)] = jnp.zeros((16,), jnp.float32)
                return c2

            return lax.fori_loop(0, 8, zcol, carry)

        lax.fori_loop(0, 128, zrow, 0)

        _for_my_slice(s, lambda off, ln: pltpu.sync_copy(
            rows.at[pl.ds(0, ln)], acc.at[pl.ds(off, ln)]))
        plsc.subcore_barrier()

        base = w * ROWS_PER_W + jnp.minimum(w, EXTRA)
        n = ROWS_PER_W + jnp.where(w < EXTRA, 1, 0)

        def body(j, carry):
            pltpu.sync_copy(src_hbm.at[base + j], sidx)
            pltpu.sync_copy(dst_hbm.at[base + j], didx)
            pltpu.async_copy(xws_hbm.at[sidx], rows, sem).wait()
            pltpu.sync_copy(rows, acc.at[didx], add=True)
            return carry

        lax.fori_loop(0, n, body, 0)
        plsc.subcore_barrier()

        def wb(off, ln):
            pltpu.sync_copy(acc.at[pl.ds(off, ln)], rows.at[pl.ds(0, ln)])
            pltpu.sync_copy(rows.at[pl.ds(0, ln)],
                            out_hbm.at[c].at[pl.ds(off, ln)])

        _for_my_slice(s, wb)

    return k(xws, src2, dst2)


R = 1000


def _mm_scale_body(x_ref, w_ref, degp_ref, o_ref):
    d = degp_ref[:, 0] + degp_ref[:, 1] + 1.0
    dinv = lax.rsqrt(d)
    xw = jnp.dot(x_ref[...], w_ref[...], preferred_element_type=jnp.float32,
                 precision=lax.Precision.HIGHEST)
    o_ref[...] = dinv[:, None] * xw


def _mm_scale(x, w, degp):
    return pl.pallas_call(
        _mm_scale_body,
        grid=(N // R,),
        in_specs=[
            pl.BlockSpec((R, D), lambda i: (i, 0)),
            pl.BlockSpec((D, D), lambda i: (0, 0)),
            pl.BlockSpec((R, NC), lambda i: (i, 0)),
        ],
        out_specs=pl.BlockSpec((R, D), lambda i: (i, 0)),
        out_shape=jax.ShapeDtypeStruct((N, D), jnp.float32),
    )(x, w, degp)


def _mid_body(p_ref, xws_ref, degp_ref, b_ref, w2_ref, o_ref):
    d = degp_ref[:, 0] + degp_ref[:, 1] + 1.0
    dinv = lax.rsqrt(d)
    ssum = p_ref[0] + p_ref[1] + xws_ref[...]
    h = jnp.maximum(dinv[:, None] * ssum + b_ref[0, :][None, :], 0.0)
    hw = jnp.dot(h, w2_ref[...], preferred_element_type=jnp.float32,
                 precision=lax.Precision.HIGHEST)
    o_ref[...] = dinv[:, None] * hw


def _mid(p, xws, degp, b1, w2):
    return pl.pallas_call(
        _mid_body,
        grid=(N // R,),
        in_specs=[
            pl.BlockSpec((NC, R, D), lambda i: (0, i, 0)),
            pl.BlockSpec((R, D), lambda i: (i, 0)),
            pl.BlockSpec((R, NC), lambda i: (i, 0)),
            pl.BlockSpec((1, D), lambda i: (0, 0)),
            pl.BlockSpec((D, D), lambda i: (0, 0)),
        ],
        out_specs=pl.BlockSpec((R, D), lambda i: (i, 0)),
        out_shape=jax.ShapeDtypeStruct((N, D), jnp.float32),
    )(p, xws, degp, b1, w2)


def _final_body(q_ref, xws2_ref, degp_ref, b_ref, o_ref):
    d = degp_ref[:, 0] + degp_ref[:, 1] + 1.0
    dinv = lax.rsqrt(d)
    ssum = q_ref[0] + q_ref[1] + xws2_ref[...]
    o_ref[...] = jnp.maximum(dinv[:, None] * ssum + b_ref[0, :][None, :], 0.0)


def _final(q, xws2, degp, b2):
    return pl.pallas_call(
        _final_body,
        grid=(N // R,),
        in_specs=[
            pl.BlockSpec((NC, R, D), lambda i: (0, i, 0)),
            pl.BlockSpec((R, D), lambda i: (i, 0)),
            pl.BlockSpec((R, NC), lambda i: (i, 0)),
            pl.BlockSpec((1, D), lambda i: (0, 0)),
        ],
        out_specs=pl.BlockSpec((R, D), lambda i: (i, 0)),
        out_shape=jax.ShapeDtypeStruct((N, D), jnp.float32),
    )(q, xws2, degp, b2)


def kernel(x, edge_index, W1, b1, W2, b2):
    src2 = edge_index[0].reshape(CH, 128)
    dst2 = edge_index[1].reshape(CH, 128)
    b1r = b1.reshape(1, D)
    b2r = b2.reshape(1, D)

    degp = _degree_partials(dst2).reshape(NC, N).T
    xws1 = _mm_scale(x, W1, degp)
    p = _propagate(xws1, src2, dst2)
    xws2 = _mid(p, xws1, degp, b1r, W2)
    q = _propagate(xws2, src2, dst2)
    return _final(q, xws2, degp, b2r)
